# baseline (speedup 1.0000x reference)
import functools

import jax
import jax.numpy as jnp
from jax import lax
from jax.experimental import pallas as pl
from jax.experimental.pallas import tpu as pltpu
from jax.experimental.pallas import tpu_sc as plsc

_D = 128
_L = 16
_NC = 2
_NS = 16
_NW = _NC * _NS
_K = 128
_IB = 8
_TB = 1000


def _make_sc_agg(n_nodes, n_edges_pad, with_deg):
    np_rows = ((n_nodes + 1 + _NS * _K - 1) // (_NS * _K)) * (_NS * _K)
    rpt = np_rows // _NS
    niter = n_edges_pad // (_NW * _K)
    assert niter % (2 * _IB) == 0
    nblk = niter // _IB

    out_types = [jax.ShapeDtypeStruct((_NC, np_rows, _D), jnp.float32)]
    scratch = [
        pltpu.VMEM((2, _IB, _K), jnp.int32),
        pltpu.VMEM((2, _IB, _K), jnp.int32),
        pltpu.VMEM((_K, _D), jnp.float32),
        pltpu.VMEM((_K, _D), jnp.float32),
        pltpu.VMEM_SHARED((np_rows, _D), jnp.float32),
        pltpu.SemaphoreType.DMA,
        pltpu.SemaphoreType.DMA,
        pltpu.SemaphoreType.DMA,
        pltpu.SemaphoreType.DMA,
    ]
    if with_deg:
        out_types.append(jax.ShapeDtypeStruct((_NC, np_rows), jnp.float32))
        scratch += [
            pltpu.VMEM((_K,), jnp.float32),
            pltpu.VMEM((_K,), jnp.float32),
            pltpu.VMEM_SHARED((np_rows,), jnp.float32),
        ]
    mesh = plsc.VectorSubcoreMesh(core_axis_name="c", subcore_axis_name="s")

    def body(x_hbm, src_hbm, dst_hbm, *rest):
        if with_deg:
            out_hbm, deg_hbm = rest[0], rest[1]
            (src_a, dst_a, rows0, rows1, acc_s, sem0, sem1, semi0, semi1,
             ones_v, zeros_v, dega_s) = rest[2:]
        else:
            out_hbm = rest[0]
            (src_a, dst_a, rows0, rows1, acc_s, sem0, sem1, semi0,
             semi1) = rest[1:]

        c = lax.axis_index("c")
        s = lax.axis_index("s")
        wid = s * _NC + c
        cbase = wid * niter

        def _zrow(r, carry):
            for cb in range(_D // _L):
                rows0[r, pl.ds(cb * _L, _L)] = jnp.zeros((_L,), jnp.float32)
            return carry
        lax.fori_loop(0, _K, _zrow, 0)
        if with_deg:
            for cb in range(_K // _L):
                ones_v[pl.ds(cb * _L, _L)] = jnp.ones((_L,), jnp.float32)
                zeros_v[pl.ds(cb * _L, _L)] = jnp.zeros((_L,), jnp.float32)

        rbase = s * rpt
        zcp = []
        for j in range(rpt // _K):
            zcp.append(pltpu.make_async_copy(
                rows0, acc_s.at[pl.ds(rbase + j * _K, _K)], sem0))
            zcp[-1].start()
            if with_deg:
                zcp.append(pltpu.make_async_copy(
                    zeros_v, dega_s.at[pl.ds(rbase + j * _K, _K)], sem1))
                zcp[-1].start()
        for cp in zcp:
            cp.wait()
        plsc.subcore_barrier()

        def _idx_fetch(b, slot):
            cb0 = cbase + b * _IB
            pltpu.make_async_copy(src_hbm.at[pl.ds(cb0, _IB)],
                                  src_a.at[slot], semi0).start()
            pltpu.make_async_copy(dst_hbm.at[pl.ds(cb0, _IB)],
                                  dst_a.at[slot], semi1).start()

        def _idx_wait(slot):
            pltpu.make_async_copy(src_hbm.at[pl.ds(cbase, _IB)],
                                  src_a.at[slot], semi0).wait()
            pltpu.make_async_copy(dst_hbm.at[pl.ds(cbase, _IB)],
                                  dst_a.at[slot], semi1).wait()

        def _process(slot):
            pltpu.make_async_copy(x_hbm.at[src_a.at[slot, 0]], rows0,
                                  sem0).start()
            for t in range(_IB // 2):
                e0 = 2 * t
                e1 = e0 + 1
                pltpu.make_async_copy(x_hbm.at[src_a.at[slot, e1]], rows1,
                                      sem1).start()
                pltpu.make_async_copy(x_hbm.at[src_a.at[slot, e0]], rows0,
                                      sem0).wait()
                pltpu.sync_copy(rows0, acc_s.at[dst_a.at[slot, e0]], add=True)
                if with_deg:
                    pltpu.sync_copy(ones_v, dega_s.at[dst_a.at[slot, e0]],
                                    add=True)
                if e0 + 2 < _IB:
                    pltpu.make_async_copy(x_hbm.at[src_a.at[slot, e0 + 2]],
                                          rows0, sem0).start()
                pltpu.make_async_copy(x_hbm.at[src_a.at[slot, e1]], rows1,
                                      sem1).wait()
                pltpu.sync_copy(rows1, acc_s.at[dst_a.at[slot, e1]], add=True)
                if with_deg:
                    pltpu.sync_copy(ones_v, dega_s.at[dst_a.at[slot, e1]],
                                    add=True)

        _idx_fetch(0, 0)
        _idx_wait(0)

        def _block_pair(bb, carry):
            b0 = 2 * bb
            _idx_fetch(b0 + 1, 1)
            _process(0)
            _idx_wait(1)

            @pl.when(b0 + 2 < nblk)
            def _():
                _idx_fetch(b0 + 2, 0)
            _process(1)

            @pl.when(b0 + 2 < nblk)
            def _():
                _idx_wait(0)
            return carry
        lax.fori_loop(0, nblk // 2, _block_pair, 0)

        plsc.subcore_barrier()
        for j in range(rpt // _K):
            r0 = rbase + j * _K
            pltpu.sync_copy(acc_s.at[pl.ds(r0, _K)], out_hbm.at[c, pl.ds(r0, _K)])
        if with_deg:
            pltpu.sync_copy(dega_s.at[pl.ds(rbase, rpt)],
                            deg_hbm.at[c, pl.ds(rbase, rpt)])

    return pl.kernel(body, mesh=mesh, out_type=out_types, scratch_types=scratch)


def _make_tc_layer(n_nodes, np_rows, relu):
    nblk = n_nodes // _TB

    def body(pa, pb, d0, d1, xr, wl, wr, br, o):
        deg = jnp.maximum(d0[...] + d1[...], 1.0)
        agg = (pa[0] + pb[0]) / deg
        acc = jnp.dot(agg, wl[...], preferred_element_type=jnp.float32)
        acc = acc + jnp.dot(xr[...], wr[...], preferred_element_type=jnp.float32)
        acc = acc + br[...]
        if relu:
            acc = jnp.maximum(acc, 0.0)
        o[...] = acc

    return pl.pallas_call(
        body,
        grid=(nblk,),
        in_specs=[
            pl.BlockSpec((1, _TB, _D), lambda i: (0, i, 0)),
            pl.BlockSpec((1, _TB, _D), lambda i: (1, i, 0)),
            pl.BlockSpec((_TB, 1), lambda i: (i, 0)),
            pl.BlockSpec((_TB, 1), lambda i: (i, 0)),
            pl.BlockSpec((_TB, _D), lambda i: (i, 0)),
            pl.BlockSpec((_D, _D), lambda i: (0, 0)),
            pl.BlockSpec((_D, _D), lambda i: (0, 0)),
            pl.BlockSpec((1, _D), lambda i: (0, 0)),
        ],
        out_specs=pl.BlockSpec((_TB, _D), lambda i: (i, 0)),
        out_shape=jax.ShapeDtypeStruct((n_nodes, _D), jnp.float32),
    )


def kernel(x, edge_index, W1_l, W1_r, b1, W2_l, W2_r, b2):
    n = x.shape[0]
    e = edge_index.shape[1]
    src = edge_index[0].astype(jnp.int32)
    dst = edge_index[1].astype(jnp.int32)
    niter = -(-e // (_NW * _K))
    niter = -(-niter // (2 * _IB)) * (2 * _IB)
    e_pad = niter * _NW * _K
    pad = e_pad - e
    np_rows = ((n + 1 + _NS * _K - 1) // (_NS * _K)) * (_NS * _K)
    if pad:
        fill = jnp.arange(pad, dtype=jnp.int32)
        src = jnp.concatenate([src, fill % n])
        dst = jnp.concatenate([dst, n + fill % (np_rows - n)])
    src = src.reshape(e_pad // _K, _K)
    dst = dst.reshape(e_pad // _K, _K)

    sc_agg_deg = _make_sc_agg(n, e_pad, True)
    sc_agg = _make_sc_agg(n, e_pad, False)
    l1 = _make_tc_layer(n, np_rows, True)
    l2 = _make_tc_layer(n, np_rows, False)

    part1, degp = sc_agg_deg(x, src, dst)
    d0 = degp[0, :n, None]
    d1 = degp[1, :n, None]
    h = l1(part1, part1, d0, d1, x, W1_l, W1_r, b1.reshape(1, _D))
    part2 = sc_agg(h, src, dst)
    if isinstance(part2, (tuple, list)):
        part2 = part2[0]
    out = l2(part2, part2, d0, d1, h, W2_l, W2_r, b2.reshape(1, _D))
    return out


# speedup vs baseline: 1.0232x; 1.0232x over previous
import functools

import jax
import jax.numpy as jnp
from jax import lax
from jax.experimental import pallas as pl
from jax.experimental.pallas import tpu as pltpu
from jax.experimental.pallas import tpu_sc as plsc

_D = 128
_L = 16
_NC = 2
_NS = 16
_NW = _NC * _NS
_K = 128
_IB = 16
_TB = 1000


def _make_sc_agg(n_nodes, n_edges_pad, with_deg):
    np_rows = ((n_nodes + 1 + _NS * _K - 1) // (_NS * _K)) * (_NS * _K)
    rpt = np_rows // _NS
    niter = n_edges_pad // (_NW * _K)
    assert niter % _IB == 0
    nblk = niter // _IB

    out_types = [jax.ShapeDtypeStruct((_NC, np_rows, _D), jnp.float32)]
    scratch = [
        pltpu.VMEM((_IB, _K), jnp.int32),
        pltpu.VMEM((_IB, _K), jnp.int32),
        pltpu.VMEM((_K, _D), jnp.float32),
        pltpu.VMEM((_K, _D), jnp.float32),
        pltpu.VMEM_SHARED((np_rows, _D), jnp.float32),
        pltpu.SemaphoreType.DMA,
        pltpu.SemaphoreType.DMA,
    ]
    if with_deg:
        out_types.append(jax.ShapeDtypeStruct((_NC, np_rows), jnp.float32))
        scratch += [
            pltpu.VMEM((_K,), jnp.float32),
            pltpu.VMEM((_K,), jnp.float32),
            pltpu.VMEM_SHARED((np_rows,), jnp.float32),
        ]
    mesh = plsc.VectorSubcoreMesh(core_axis_name="c", subcore_axis_name="s")

    def body(x_hbm, src_hbm, dst_hbm, *rest):
        if with_deg:
            out_hbm, deg_hbm = rest[0], rest[1]
            (src_a, dst_a, rows0, rows1, acc_s, sem0, sem1,
             ones_v, zeros_v, dega_s) = rest[2:]
        else:
            out_hbm = rest[0]
            src_a, dst_a, rows0, rows1, acc_s, sem0, sem1 = rest[1:]

        c = lax.axis_index("c")
        s = lax.axis_index("s")
        wid = s * _NC + c
        cbase = wid * niter

        def _zrow(r, carry):
            for cb in range(_D // _L):
                rows0[r, pl.ds(cb * _L, _L)] = jnp.zeros((_L,), jnp.float32)
            return carry
        lax.fori_loop(0, _K, _zrow, 0)
        if with_deg:
            for cb in range(_K // _L):
                ones_v[pl.ds(cb * _L, _L)] = jnp.ones((_L,), jnp.float32)
                zeros_v[pl.ds(cb * _L, _L)] = jnp.zeros((_L,), jnp.float32)

        rbase = s * rpt
        zcp = []
        for j in range(rpt // _K):
            zcp.append(pltpu.make_async_copy(
                rows0, acc_s.at[pl.ds(rbase + j * _K, _K)], sem0))
            zcp[-1].start()
            if with_deg:
                zcp.append(pltpu.make_async_copy(
                    zeros_v, dega_s.at[pl.ds(rbase + j * _K, _K)], sem1))
                zcp[-1].start()
        for cp in zcp:
            cp.wait()
        plsc.subcore_barrier()

        def _block(b, carry):
            cb0 = cbase + b * _IB
            cps = pltpu.make_async_copy(src_hbm.at[pl.ds(cb0, _IB)], src_a,
                                        sem0)
            cpd = pltpu.make_async_copy(dst_hbm.at[pl.ds(cb0, _IB)], dst_a,
                                        sem1)
            cps.start()
            cpd.start()
            cps.wait()
            cpd.wait()
            pltpu.make_async_copy(x_hbm.at[src_a.at[0]], rows0, sem0).start()
            for t in range(_IB // 2):
                e0 = 2 * t
                e1 = e0 + 1
                pltpu.make_async_copy(x_hbm.at[src_a.at[e1]], rows1,
                                      sem1).start()
                pltpu.make_async_copy(x_hbm.at[src_a.at[e0]], rows0,
                                      sem0).wait()
                pltpu.sync_copy(rows0, acc_s.at[dst_a.at[e0]], add=True)
                if with_deg:
                    pltpu.sync_copy(ones_v, dega_s.at[dst_a.at[e0]], add=True)
                if e0 + 2 < _IB:
                    pltpu.make_async_copy(x_hbm.at[src_a.at[e0 + 2]], rows0,
                                          sem0).start()
                pltpu.make_async_copy(x_hbm.at[src_a.at[e1]], rows1,
                                      sem1).wait()
                pltpu.sync_copy(rows1, acc_s.at[dst_a.at[e1]], add=True)
                if with_deg:
                    pltpu.sync_copy(ones_v, dega_s.at[dst_a.at[e1]], add=True)
            return carry
        lax.fori_loop(0, nblk, _block, 0)

        plsc.subcore_barrier()
        for j in range(rpt // _K):
            r0 = rbase + j * _K
            pltpu.sync_copy(acc_s.at[pl.ds(r0, _K)], out_hbm.at[c, pl.ds(r0, _K)])
        if with_deg:
            pltpu.sync_copy(dega_s.at[pl.ds(rbase, rpt)],
                            deg_hbm.at[c, pl.ds(rbase, rpt)])

    return pl.kernel(body, mesh=mesh, out_type=out_types, scratch_types=scratch)


def _make_tc_layer(n_nodes, np_rows, relu):
    nblk = n_nodes // _TB

    def body(pa, pb, d0, d1, xr, wl, wr, br, o):
        deg = jnp.maximum(d0[...] + d1[...], 1.0)
        agg = (pa[0] + pb[0]) / deg
        acc = jnp.dot(agg, wl[...], preferred_element_type=jnp.float32)
        acc = acc + jnp.dot(xr[...], wr[...], preferred_element_type=jnp.float32)
        acc = acc + br[...]
        if relu:
            acc = jnp.maximum(acc, 0.0)
        o[...] = acc

    return pl.pallas_call(
        body,
        grid=(nblk,),
        in_specs=[
            pl.BlockSpec((1, _TB, _D), lambda i: (0, i, 0)),
            pl.BlockSpec((1, _TB, _D), lambda i: (1, i, 0)),
            pl.BlockSpec((_TB, 1), lambda i: (i, 0)),
            pl.BlockSpec((_TB, 1), lambda i: (i, 0)),
            pl.BlockSpec((_TB, _D), lambda i: (i, 0)),
            pl.BlockSpec((_D, _D), lambda i: (0, 0)),
            pl.BlockSpec((_D, _D), lambda i: (0, 0)),
            pl.BlockSpec((1, _D), lambda i: (0, 0)),
        ],
        out_specs=pl.BlockSpec((_TB, _D), lambda i: (i, 0)),
        out_shape=jax.ShapeDtypeStruct((n_nodes, _D), jnp.float32),
    )


def kernel(x, edge_index, W1_l, W1_r, b1, W2_l, W2_r, b2):
    n = x.shape[0]
    e = edge_index.shape[1]
    src = edge_index[0].astype(jnp.int32)
    dst = edge_index[1].astype(jnp.int32)
    niter = -(-e // (_NW * _K))
    niter = -(-niter // _IB) * _IB
    e_pad = niter * _NW * _K
    pad = e_pad - e
    np_rows = ((n + 1 + _NS * _K - 1) // (_NS * _K)) * (_NS * _K)
    if pad:
        fill = jnp.arange(pad, dtype=jnp.int32)
        src = jnp.concatenate([src, fill % n])
        dst = jnp.concatenate([dst, n + fill % (np_rows - n)])
    src = src.reshape(e_pad // _K, _K)
    dst = dst.reshape(e_pad // _K, _K)

    sc_agg_deg = _make_sc_agg(n, e_pad, True)
    sc_agg = _make_sc_agg(n, e_pad, False)
    l1 = _make_tc_layer(n, np_rows, True)
    l2 = _make_tc_layer(n, np_rows, False)

    part1, degp = sc_agg_deg(x, src, dst)
    d0 = degp[0, :n, None]
    d1 = degp[1, :n, None]
    h = l1(part1, part1, d0, d1, x, W1_l, W1_r, b1.reshape(1, _D))
    part2 = sc_agg(h, src, dst)
    if isinstance(part2, (tuple, list)):
        part2 = part2[0]
    out = l2(part2, part2, d0, d1, h, W2_l, W2_r, b2.reshape(1, _D))
    return out


# speedup vs baseline: 1.0421x; 1.0185x over previous
import functools

import jax
import jax.numpy as jnp
from jax import lax
from jax.experimental import pallas as pl
from jax.experimental.pallas import tpu as pltpu
from jax.experimental.pallas import tpu_sc as plsc

_D = 128
_L = 16
_NC = 2
_NS = 16
_NW = _NC * _NS
_K = 128
_IB = 16
_TB = 2000


def _make_sc_agg(n_nodes, n_edges_pad, with_deg):
    np_rows = ((n_nodes + 1 + _NS * _K - 1) // (_NS * _K)) * (_NS * _K)
    rpt = np_rows // _NS
    niter = n_edges_pad // (_NW * _K)
    assert niter % _IB == 0
    nblk = niter // _IB

    out_types = [jax.ShapeDtypeStruct((_NC, np_rows, _D), jnp.float32)]
    scratch = [
        pltpu.VMEM((_IB, _K), jnp.int32),
        pltpu.VMEM((_IB, _K), jnp.int32),
        pltpu.VMEM((_K, _D), jnp.float32),
        pltpu.VMEM((_K, _D), jnp.float32),
        pltpu.VMEM_SHARED((np_rows, _D), jnp.float32),
        pltpu.SemaphoreType.DMA,
        pltpu.SemaphoreType.DMA,
    ]
    if with_deg:
        out_types.append(jax.ShapeDtypeStruct((_NC, np_rows), jnp.float32))
        scratch += [
            pltpu.VMEM((_K,), jnp.float32),
            pltpu.VMEM((_K,), jnp.float32),
            pltpu.VMEM_SHARED((np_rows,), jnp.float32),
        ]
    mesh = plsc.VectorSubcoreMesh(core_axis_name="c", subcore_axis_name="s")

    def body(x_hbm, src_hbm, dst_hbm, *rest):
        if with_deg:
            out_hbm, deg_hbm = rest[0], rest[1]
            (src_a, dst_a, rows0, rows1, acc_s, sem0, sem1,
             ones_v, zeros_v, dega_s) = rest[2:]
        else:
            out_hbm = rest[0]
            src_a, dst_a, rows0, rows1, acc_s, sem0, sem1 = rest[1:]

        c = lax.axis_index("c")
        s = lax.axis_index("s")
        wid = s * _NC + c
        cbase = wid * niter

        def _zrow(r, carry):
            for cb in range(_D // _L):
                rows0[r, pl.ds(cb * _L, _L)] = jnp.zeros((_L,), jnp.float32)
            return carry
        lax.fori_loop(0, _K, _zrow, 0)
        if with_deg:
            for cb in range(_K // _L):
                ones_v[pl.ds(cb * _L, _L)] = jnp.ones((_L,), jnp.float32)
                zeros_v[pl.ds(cb * _L, _L)] = jnp.zeros((_L,), jnp.float32)

        rbase = s * rpt
        zcp = []
        for j in range(rpt // _K):
            zcp.append(pltpu.make_async_copy(
                rows0, acc_s.at[pl.ds(rbase + j * _K, _K)], sem0))
            zcp[-1].start()
            if with_deg:
                zcp.append(pltpu.make_async_copy(
                    zeros_v, dega_s.at[pl.ds(rbase + j * _K, _K)], sem1))
                zcp[-1].start()
        for cp in zcp:
            cp.wait()
        plsc.subcore_barrier()

        def _block(b, carry):
            cb0 = cbase + b * _IB
            cps = pltpu.make_async_copy(src_hbm.at[pl.ds(cb0, _IB)], src_a,
                                        sem0)
            cpd = pltpu.make_async_copy(dst_hbm.at[pl.ds(cb0, _IB)], dst_a,
                                        sem1)
            cps.start()
            cpd.start()
            cps.wait()
            cpd.wait()
            pltpu.make_async_copy(x_hbm.at[src_a.at[0]], rows0, sem0).start()
            for t in range(_IB // 2):
                e0 = 2 * t
                e1 = e0 + 1
                pltpu.make_async_copy(x_hbm.at[src_a.at[e1]], rows1,
                                      sem1).start()
                pltpu.make_async_copy(x_hbm.at[src_a.at[e0]], rows0,
                                      sem0).wait()
                pltpu.sync_copy(rows0, acc_s.at[dst_a.at[e0]], add=True)
                if with_deg:
                    pltpu.sync_copy(ones_v, dega_s.at[dst_a.at[e0]], add=True)
                if e0 + 2 < _IB:
                    pltpu.make_async_copy(x_hbm.at[src_a.at[e0 + 2]], rows0,
                                          sem0).start()
                pltpu.make_async_copy(x_hbm.at[src_a.at[e1]], rows1,
                                      sem1).wait()
                pltpu.sync_copy(rows1, acc_s.at[dst_a.at[e1]], add=True)
                if with_deg:
                    pltpu.sync_copy(ones_v, dega_s.at[dst_a.at[e1]], add=True)
            return carry
        lax.fori_loop(0, nblk, _block, 0)

        plsc.subcore_barrier()
        for j in range(rpt // _K):
            r0 = rbase + j * _K
            pltpu.sync_copy(acc_s.at[pl.ds(r0, _K)], out_hbm.at[c, pl.ds(r0, _K)])
        if with_deg:
            pltpu.sync_copy(dega_s.at[pl.ds(rbase, rpt)],
                            deg_hbm.at[c, pl.ds(rbase, rpt)])

    return pl.kernel(body, mesh=mesh, out_type=out_types, scratch_types=scratch)


def _make_tc_layer(n_nodes, np_rows, relu):
    nblk = n_nodes // _TB

    def body(pa, pb, d0, d1, xr, wl, wr, br, o):
        deg = jnp.maximum(d0[...] + d1[...], 1.0)
        agg = (pa[0] + pb[0]) / deg
        acc = jnp.dot(agg, wl[...], preferred_element_type=jnp.float32)
        acc = acc + jnp.dot(xr[...], wr[...], preferred_element_type=jnp.float32)
        acc = acc + br[...]
        if relu:
            acc = jnp.maximum(acc, 0.0)
        o[...] = acc

    return pl.pallas_call(
        body,
        grid=(nblk,),
        in_specs=[
            pl.BlockSpec((1, _TB, _D), lambda i: (0, i, 0)),
            pl.BlockSpec((1, _TB, _D), lambda i: (1, i, 0)),
            pl.BlockSpec((_TB, 1), lambda i: (i, 0)),
            pl.BlockSpec((_TB, 1), lambda i: (i, 0)),
            pl.BlockSpec((_TB, _D), lambda i: (i, 0)),
            pl.BlockSpec((_D, _D), lambda i: (0, 0)),
            pl.BlockSpec((_D, _D), lambda i: (0, 0)),
            pl.BlockSpec((1, _D), lambda i: (0, 0)),
        ],
        out_specs=pl.BlockSpec((_TB, _D), lambda i: (i, 0)),
        out_shape=jax.ShapeDtypeStruct((n_nodes, _D), jnp.float32),
    )


def kernel(x, edge_index, W1_l, W1_r, b1, W2_l, W2_r, b2):
    n = x.shape[0]
    e = edge_index.shape[1]
    src = edge_index[0].astype(jnp.int32)
    dst = edge_index[1].astype(jnp.int32)
    niter = -(-e // (_NW * _K))
    niter = -(-niter // _IB) * _IB
    e_pad = niter * _NW * _K
    pad = e_pad - e
    np_rows = ((n + 1 + _NS * _K - 1) // (_NS * _K)) * (_NS * _K)
    if pad:
        fill = jnp.arange(pad, dtype=jnp.int32)
        src = jnp.concatenate([src, fill % n])
        dst = jnp.concatenate([dst, n + fill % (np_rows - n)])
    src = src.reshape(e_pad // _K, _K)
    dst = dst.reshape(e_pad // _K, _K)

    sc_agg_deg = _make_sc_agg(n, e_pad, True)
    sc_agg = _make_sc_agg(n, e_pad, False)
    l1 = _make_tc_layer(n, np_rows, True)
    l2 = _make_tc_layer(n, np_rows, False)

    part1, degp = sc_agg_deg(x, src, dst)
    d0 = degp[0, :n, None]
    d1 = degp[1, :n, None]
    h = l1(part1, part1, d0, d1, x, W1_l, W1_r, b1.reshape(1, _D))
    part2 = sc_agg(h, src, dst)
    if isinstance(part2, (tuple, list)):
        part2 = part2[0]
    out = l2(part2, part2, d0, d1, h, W2_l, W2_r, b2.reshape(1, _D))
    return out
